# Initial kernel scaffold; baseline (speedup 1.0000x reference)
import jax
import jax.numpy as jnp
from jax.experimental import pallas as pl


def kernel(h, edge_attr, edges, node_mask, edge_mask, W, bias, att_w1, att_b1, att_w2, att_b2, ln_gamma, ln_beta):
    raise NotImplementedError("write your pallas kernel here")


# speedup vs baseline: 3.2327x; 3.2327x over previous
import functools

import jax
import jax.numpy as jnp
from jax import lax
from jax.experimental import pallas as pl
from jax.experimental.pallas import tpu as pltpu
from jax.experimental.pallas import tpu_sc as plsc

N = 10000
E = 320000
D = 128

_F32 = jnp.float32


def _acosh(z):
    return jnp.log(z + jnp.sqrt(z * z - 1.0))


def _cosh_sinh(n):
    e = jnp.exp(n)
    ei = 1.0 / e
    return 0.5 * (e + ei), 0.5 * (e - ei)


def _sigmoid(z):
    return 1.0 / (1.0 + jnp.exp(-z))


def _rowsum(z):
    return jnp.sum(z, axis=-1, keepdims=True)


def _first_mask(shape):
    return lax.broadcasted_iota(jnp.int32, shape, len(shape) - 1) == 0


def _logmap0_cols(x, first):
    x0 = jnp.maximum(x[:, 0:1], 1.0 + 1e-7)
    d = _acosh(x0)
    nsq = _rowsum(x * x) - x[:, 0:1] * x[:, 0:1]
    n = jnp.sqrt(jnp.maximum(nsq, 1e-12))
    return jnp.where(first, 0.0, (d / n) * x)


def _expmap0_cols(u, first):
    us = jnp.where(first, 0.0, u)
    nsq = _rowsum(us * us)
    n = jnp.sqrt(jnp.maximum(nsq, 1e-12))
    c, s = _cosh_sinh(n)
    return jnp.where(first, c, (s / n) * us)


def _prologue_body(h_ref, wt_ref, bias_ref, w1at_ref, w1bt_ref,
                   x_ref, tr_ref, tc_ref):
    h = h_ref[...]
    first = _first_mask(h.shape)
    u = _logmap0_cols(h, first)
    xt = jnp.dot(u, wt_ref[...], preferred_element_type=_F32)
    xt = jnp.where(first, 0.0, xt)
    x = _expmap0_cols(xt, first)
    bmask = jnp.where(_first_mask(bias_ref[...].shape), 0.0, bias_ref[...])
    c = _rowsum(x * bmask) / (1.0 + x[:, 0:1])
    b = bmask + c * (jnp.where(first, 1.0, 0.0) + x)
    lbb = _rowsum(b * b) - 2.0 * b[:, 0:1] * b[:, 0:1]
    nb = jnp.sqrt(jnp.maximum(lbb, 1e-12))
    ch, sh = _cosh_sinh(nb)
    x = ch * x + (sh / nb) * b
    x_ref[...] = x
    x_tan = _logmap0_cols(x, first)
    a = jnp.dot(x_tan, w1at_ref[...], preferred_element_type=_F32)
    bb = jnp.dot(x_tan, w1bt_ref[...], preferred_element_type=_F32)
    tr_ref[:, 0:D] = x
    tr_ref[:, D:2 * D] = a
    tc_ref[:, 0:D] = x
    tc_ref[:, D:2 * D] = bb


def _edge_body(gr_ref, gc_ref, ea_ref, em_ref, wv_ref, agg_ref):
    xr = gr_ref[:, 0:D]
    ar = gr_ref[:, D:2 * D]
    xc = gc_ref[:, 0:D]
    bc = gc_ref[:, D:2 * D]
    alpha = -( _rowsum(xr * xc) - 2.0 * xr[:, 0:1] * xc[:, 0:1])
    alpha = jnp.maximum(alpha, 1.0 + 1e-7)
    d = _acosh(alpha)
    beta = jnp.sqrt(jnp.maximum(alpha * alpha - 1.0, 1e-12))
    w1c0 = wv_ref[0:1, :]
    w1c1 = wv_ref[1:2, :]
    b1 = wv_ref[2:3, :]
    w2 = wv_ref[3:4, :]
    b2 = wv_ref[4:5, 0:1]
    pre = ar + bc + ea_ref[...] * w1c0 + d * w1c1 + b1
    hmid = pre * _sigmoid(pre)
    att = _sigmoid(_rowsum(hmid * w2) + b2) * em_ref[...]
    agg_ref[...] = ((d / beta) * att) * (xc - alpha * xr)


def _epilogue_body(x_ref, o0_ref, o1_ref, ln_ref, out_ref):
    x = x_ref[...]
    first = _first_mask(x.shape)
    out = (o0_ref[...] + o1_ref[...]) * (1.0 / 1000.0)
    lxo = _rowsum(x * out) - 2.0 * x[:, 0:1] * out[:, 0:1]
    p = out + lxo * x
    lpp = _rowsum(p * p) - 2.0 * p[:, 0:1] * p[:, 0:1]
    npn = jnp.sqrt(jnp.maximum(lpp, 1e-12))
    ch, sh = _cosh_sinh(npn)
    x2 = ch * x + (sh / npn) * p
    ht = _logmap0_cols(x2, first)
    mu = _rowsum(ht) * (1.0 / (D - 1))
    dif = jnp.where(first, 0.0, ht - mu)
    var = _rowsum(dif * dif) * (1.0 / (D - 1))
    gamma = ln_ref[0:1, :]
    beta = ln_ref[1:2, :]
    h1 = dif / jnp.sqrt(var + 1e-5) * gamma + beta
    x3 = _expmap0_cols(h1, first)
    s = _logmap0_cols(x3, first)
    sl = s * _sigmoid(s)
    out_ref[...] = _expmap0_cols(sl, first)


_NC = 2
_NS = 16
_NW = _NC * _NS
_PERW = E // _NW
_CH = 80
_NCHUNK = _PERW // _CH
_NPAD = 10240
_ROWS_PER_TILE = _NPAD // _NS
_ZR = 128


def _sc_gather(tr, tcb, row2d, col2d):
    mesh = plsc.VectorSubcoreMesh(core_axis_name="c", subcore_axis_name="s")

    @functools.partial(
        pl.kernel, mesh=mesh,
        out_type=[jax.ShapeDtypeStruct((E, 2 * D), _F32),
                  jax.ShapeDtypeStruct((E, 2 * D), _F32)],
        scratch_types=[
            pltpu.VMEM((_NCHUNK, _CH), jnp.int32),
            pltpu.VMEM((_NCHUNK, _CH), jnp.int32),
            pltpu.VMEM((_CH, 2 * D), _F32),
            pltpu.VMEM((_CH, 2 * D), _F32),
            pltpu.SemaphoreType.DMA,
            pltpu.SemaphoreType.DMA,
        ],
    )
    def k(tr_hbm, tc_hbm, row_hbm, col_hbm, gr_hbm, gc_hbm,
          idxr, idxc, bufr, bufc, semr, semc):
        cid = lax.axis_index("c")
        sid = lax.axis_index("s")
        wid = sid * _NC + cid
        pltpu.sync_copy(row_hbm.at[wid], idxr)
        pltpu.sync_copy(col_hbm.at[wid], idxc)

        def body(j, carry):
            ebase = pl.multiple_of(wid * _PERW + j * _CH, 8)
            cr = pltpu.async_copy(tr_hbm.at[idxr.at[j]], bufr, semr)
            cr.wait()
            pltpu.sync_copy(bufr, gr_hbm.at[pl.ds(ebase, _CH)])
            cc = pltpu.async_copy(tc_hbm.at[idxc.at[j]], bufc, semc)
            cc.wait()
            pltpu.sync_copy(bufc, gc_hbm.at[pl.ds(ebase, _CH)])
            return carry

        lax.fori_loop(0, _NCHUNK, body, 0)

    return k(tr, tcb, row2d, col2d)


def _sc_scatter(agg, row2d, zeros_hbm):
    mesh = plsc.VectorSubcoreMesh(core_axis_name="c", subcore_axis_name="s")

    @functools.partial(
        pl.kernel, mesh=mesh,
        out_type=jax.ShapeDtypeStruct((_NC, _NPAD, D), _F32),
        scratch_types=[
            pltpu.VMEM((_NCHUNK, _CH), jnp.int32),
            pltpu.VMEM((_CH, D), _F32),
            pltpu.VMEM((_ZR, D), _F32),
            pltpu.VMEM_SHARED((_NPAD, D), _F32),
            pltpu.SemaphoreType.DMA,
        ],
    )
    def k(agg_hbm, row_hbm, zeros_h, parts_hbm, idxr, buf, zbuf, acc, sem):
        cid = lax.axis_index("c")
        sid = lax.axis_index("s")
        wid = sid * _NC + cid
        pltpu.sync_copy(zeros_h, zbuf)

        def zbody(t, carry):
            rbase = pl.multiple_of(sid * _ROWS_PER_TILE + t * _ZR, 8)
            pltpu.sync_copy(zbuf, acc.at[pl.ds(rbase, _ZR)])
            return carry

        lax.fori_loop(0, _ROWS_PER_TILE // _ZR, zbody, 0)
        plsc.subcore_barrier()

        pltpu.sync_copy(row_hbm.at[wid], idxr)

        def body(j, carry):
            ebase = pl.multiple_of(wid * _PERW + j * _CH, 8)
            pltpu.sync_copy(agg_hbm.at[pl.ds(ebase, _CH)], buf)
            pltpu.sync_copy(buf, acc.at[idxr.at[j]], add=True)
            return carry

        lax.fori_loop(0, _NCHUNK, body, 0)
        plsc.subcore_barrier()

        def wbody(t, carry):
            rbase = pl.multiple_of(sid * _ROWS_PER_TILE + t * _ZR, 8)
            pltpu.sync_copy(acc.at[pl.ds(rbase, _ZR)], zbuf)
            pltpu.sync_copy(zbuf, parts_hbm.at[cid].at[pl.ds(rbase, _ZR)])
            return carry

        lax.fori_loop(0, _ROWS_PER_TILE // _ZR, wbody, 0)

    return k(agg, row2d, zeros_hbm)


_BN = 2000
_BE = 2000


def kernel(h, edge_attr, edges, node_mask, edge_mask, W, bias, att_w1,
           att_b1, att_w2, att_b2, ln_gamma, ln_beta):
    del node_mask
    f32 = _F32
    h = h.astype(f32)
    row = edges[0].astype(jnp.int32)
    col = edges[1].astype(jnp.int32)
    row2d = row.reshape(_NW, _NCHUNK, _CH)
    col2d = col.reshape(_NW, _NCHUNK, _CH)

    wt = W.T.astype(f32)
    w1at = att_w1[:, 0:D].T.astype(f32)
    w1bt = att_w1[:, D:2 * D].T.astype(f32)
    bias2 = bias.astype(f32).reshape(1, D)

    wv = jnp.zeros((8, D), f32)
    wv = wv.at[0].set(att_w1[:, 2 * D])
    wv = wv.at[1].set(att_w1[:, 2 * D + 1])
    wv = wv.at[2].set(att_b1)
    wv = wv.at[3].set(att_w2[0])
    wv = wv.at[4, 0].set(att_b2[0])

    ln = jnp.zeros((2, D), f32)
    ln = ln.at[0, 1:].set(ln_gamma)
    ln = ln.at[1, 1:].set(ln_beta)

    nblk = N // _BN
    x, tr, tcb = pl.pallas_call(
        _prologue_body,
        grid=(nblk,),
        in_specs=[
            pl.BlockSpec((_BN, D), lambda i: (i, 0)),
            pl.BlockSpec((D, D), lambda i: (0, 0)),
            pl.BlockSpec((1, D), lambda i: (0, 0)),
            pl.BlockSpec((D, D), lambda i: (0, 0)),
            pl.BlockSpec((D, D), lambda i: (0, 0)),
        ],
        out_specs=[
            pl.BlockSpec((_BN, D), lambda i: (i, 0)),
            pl.BlockSpec((_BN, 2 * D), lambda i: (i, 0)),
            pl.BlockSpec((_BN, 2 * D), lambda i: (i, 0)),
        ],
        out_shape=[
            jax.ShapeDtypeStruct((N, D), f32),
            jax.ShapeDtypeStruct((N, 2 * D), f32),
            jax.ShapeDtypeStruct((N, 2 * D), f32),
        ],
    )(h, wt, bias2, w1at, w1bt)

    gr, gc = _sc_gather(tr, tcb, row2d, col2d)

    eblk = E // _BE
    agg = pl.pallas_call(
        _edge_body,
        grid=(eblk,),
        in_specs=[
            pl.BlockSpec((_BE, 2 * D), lambda i: (i, 0)),
            pl.BlockSpec((_BE, 2 * D), lambda i: (i, 0)),
            pl.BlockSpec((_BE, 1), lambda i: (i, 0)),
            pl.BlockSpec((_BE, 1), lambda i: (i, 0)),
            pl.BlockSpec((8, D), lambda i: (0, 0)),
        ],
        out_specs=pl.BlockSpec((_BE, D), lambda i: (i, 0)),
        out_shape=jax.ShapeDtypeStruct((E, D), f32),
    )(gr, gc, edge_attr.astype(f32), edge_mask.astype(f32), wv)

    zeros_h = jnp.zeros((_ZR, D), f32)
    parts = _sc_scatter(agg, row2d, zeros_h)
    p0 = parts[0, :N]
    p1 = parts[1, :N]

    out = pl.pallas_call(
        _epilogue_body,
        grid=(nblk,),
        in_specs=[
            pl.BlockSpec((_BN, D), lambda i: (i, 0)),
            pl.BlockSpec((_BN, D), lambda i: (i, 0)),
            pl.BlockSpec((_BN, D), lambda i: (i, 0)),
            pl.BlockSpec((2, D), lambda i: (0, 0)),
        ],
        out_specs=pl.BlockSpec((_BN, D), lambda i: (i, 0)),
        out_shape=jax.ShapeDtypeStruct((N, D), f32),
    )(x, p0, p1, ln)

    return out
